# Initial kernel scaffold; baseline (speedup 1.0000x reference)
import jax
import jax.numpy as jnp
from jax import lax
from jax.experimental import pallas as pl
from jax.experimental.pallas import tpu as pltpu
from jax.experimental.pallas import tpu_sc as plsc

NC = 2
NS = 16
NW = NC * NS
K = 125
DW = 16


def _make_agg(N, D, n_chunks, compute_deg):
    rows_pt = N // NS

    mesh = plsc.VectorSubcoreMesh(
        core_axis_name="c", subcore_axis_name="s", num_cores=NC, num_subcores=NS
    )

    out_type = [jax.ShapeDtypeStruct((NC, N, D), jnp.float32)]
    scratch = [
        pltpu.VMEM((K, D), jnp.float32),
        pltpu.VMEM((n_chunks, K), jnp.int32),
        pltpu.VMEM((n_chunks, K), jnp.int32),
        pltpu.VMEM_SHARED((N, D), jnp.float32),
    ]
    if compute_deg:
        out_type.append(jax.ShapeDtypeStruct((NC, N, DW), jnp.float32))
        scratch.append(pltpu.VMEM((K, DW), jnp.float32))
        scratch.append(pltpu.VMEM_SHARED((N, DW), jnp.float32))

    def body(feat_hbm, src_hbm, dst_hbm, out_hbm, *rest):
        if compute_deg:
            deg_hbm, rows_v, src_v, dst_v, agg_sh, ones_v, deg_sh = rest
        else:
            rows_v, src_v, dst_v, agg_sh = rest
        c = lax.axis_index("c")
        s = lax.axis_index("s")
        wid = c * NS + s
        base = s * rows_pt

        @pl.loop(0, K)
        def _(r):
            @pl.loop(0, D, step=16)
            def _(cc):
                rows_v[r, pl.ds(cc, 16)] = jnp.zeros((16,), jnp.float32)

        @pl.loop(0, rows_pt, step=K)
        def _(r0):
            pltpu.sync_copy(rows_v, agg_sh.at[pl.ds(base + r0, K)])

        if compute_deg:
            @pl.loop(0, K)
            def _(r):
                ones_v[r, :] = jnp.zeros((DW,), jnp.float32)

            @pl.loop(0, rows_pt, step=K)
            def _(r0):
                pltpu.sync_copy(ones_v, deg_sh.at[pl.ds(base + r0, K)])

            @pl.loop(0, K)
            def _(r):
                ones_v[r, :] = jnp.ones((DW,), jnp.float32)

        pltpu.sync_copy(src_hbm.at[wid], src_v)
        pltpu.sync_copy(dst_hbm.at[wid], dst_v)
        plsc.subcore_barrier()

        @pl.loop(0, n_chunks)
        def _(j):
            pltpu.sync_copy(feat_hbm.at[src_v.at[j]], rows_v)
            pltpu.sync_copy(rows_v, agg_sh.at[dst_v.at[j]], add=True)
            if compute_deg:
                pltpu.sync_copy(ones_v, deg_sh.at[dst_v.at[j]], add=True)

        plsc.subcore_barrier()
        pltpu.sync_copy(
            agg_sh.at[pl.ds(base, rows_pt)], out_hbm.at[c, pl.ds(base, rows_pt)]
        )
        if compute_deg:
            pltpu.sync_copy(
                deg_sh.at[pl.ds(base, rows_pt)], deg_hbm.at[c, pl.ds(base, rows_pt)]
            )

    return pl.kernel(body, out_type=out_type, mesh=mesh, scratch_types=scratch)


def _make_proj(N, D, BN):
    def body(a_ref, d_ref, w_ref, b_ref, o_ref):
        a = a_ref[0] + a_ref[1]
        deg = d_ref[0, :, :1] + d_ref[1, :, :1]
        inv = 1.0 / jnp.maximum(deg, 1.0)
        h = jnp.dot(a * inv, w_ref[...], preferred_element_type=jnp.float32)
        o_ref[...] = jnp.maximum(h + b_ref[...], 0.0)

    return pl.pallas_call(
        body,
        grid=(N // BN,),
        in_specs=[
            pl.BlockSpec((NC, BN, D), lambda i: (0, i, 0)),
            pl.BlockSpec((NC, BN, DW), lambda i: (0, i, 0)),
            pl.BlockSpec((D, D), lambda i: (0, 0)),
            pl.BlockSpec((1, D), lambda i: (0, 0)),
        ],
        out_specs=pl.BlockSpec((BN, D), lambda i: (i, 0)),
        out_shape=jax.ShapeDtypeStruct((N, D), jnp.float32),
    )


@jax.jit
def kernel(x, edge_index, W1, b1, W2, b2):
    N, D = x.shape
    E = edge_index.shape[1]
    assert E % (NW * K) == 0 and N % (NS * K) == 0
    n_chunks = E // (NW * K)

    src3 = edge_index[0].astype(jnp.int32).reshape(NW, n_chunks, K)
    dst3 = edge_index[1].astype(jnp.int32).reshape(NW, n_chunks, K)

    agg_deg = _make_agg(N, D, n_chunks, True)
    agg_only = _make_agg(N, D, n_chunks, False)
    proj1 = _make_proj(N, W1.shape[1], 1000)
    proj2 = _make_proj(N, W2.shape[1], 1000)

    agg1, deg = agg_deg(x, src3, dst3)
    h1 = proj1(agg1, deg, W1, b1.reshape(1, -1))
    (agg2,) = agg_only(h1, src3, dst3)
    h2 = proj2(agg2, deg, W2, b2.reshape(1, -1))
    return h2


# speedup vs baseline: 4.4257x; 4.4257x over previous
import dataclasses

import jax
import jax.numpy as jnp
from jax import lax
from jax.experimental import pallas as pl
from jax.experimental.pallas import tpu as pltpu
from jax.experimental.pallas import tpu_sc as plsc

NC = 2
NS = 16
NW = NC * NS
K = 128


def _make_agg(NP, D, n_chunks, compute_deg):
    rows_pt = NP // NS

    mesh = plsc.VectorSubcoreMesh(
        core_axis_name="c", subcore_axis_name="s", num_cores=NC, num_subcores=NS
    )

    out_type = [jax.ShapeDtypeStruct((NC, NP, D), jnp.float32)]
    scratch = [
        pltpu.VMEM((K, D), jnp.float32),
        pltpu.VMEM((K,), jnp.int32),
        pltpu.VMEM((K,), jnp.int32),
        pltpu.VMEM_SHARED((NP, D), jnp.float32),
    ]
    if compute_deg:
        out_type.append(jax.ShapeDtypeStruct((NW * NP,), jnp.float32))
        scratch.append(pltpu.VMEM((NP,), jnp.float32))

    def body(feat_hbm, src_hbm, dst_hbm, out_hbm, *rest):
        if compute_deg:
            deg_hbm, rows_v, src_v, dst_v, agg_sh, deg_v = rest
        else:
            rows_v, src_v, dst_v, agg_sh = rest
        c = lax.axis_index("c")
        s = lax.axis_index("s")
        wid = c * NS + s
        base = s * rows_pt

        @pl.loop(0, K)
        def _(r):
            @pl.loop(0, D, step=16)
            def _(cc):
                rows_v[r, pl.ds(cc, 16)] = jnp.zeros((16,), jnp.float32)

        @pl.loop(0, rows_pt, step=K)
        def _(r0):
            pltpu.sync_copy(rows_v, agg_sh.at[pl.ds(base + r0, K)])

        if compute_deg:
            @pl.loop(0, NP, step=16)
            def _(r0):
                deg_v[pl.ds(r0, 16)] = jnp.zeros((16,), jnp.float32)

        plsc.subcore_barrier()

        ebase = wid * (n_chunks * K)

        @pl.loop(0, n_chunks)
        def _(j):
            pltpu.sync_copy(src_hbm.at[pl.ds(ebase + j * K, K)], src_v)
            pltpu.sync_copy(dst_hbm.at[pl.ds(ebase + j * K, K)], dst_v)
            pltpu.sync_copy(feat_hbm.at[src_v], rows_v)
            pltpu.sync_copy(rows_v, agg_sh.at[dst_v], add=True)
            if compute_deg:
                @pl.loop(0, K, step=16)
                def _(t):
                    idx = dst_v[pl.ds(t, 16)]
                    plsc.addupdate_scatter(
                        deg_v, [idx], jnp.ones((16,), jnp.float32)
                    )

        plsc.subcore_barrier()
        pltpu.sync_copy(
            agg_sh.at[pl.ds(base, rows_pt)], out_hbm.at[c, pl.ds(base, rows_pt)]
        )
        if compute_deg:
            pltpu.sync_copy(deg_v, deg_hbm.at[pl.ds(wid * NP, NP)])

    cp = pltpu.CompilerParams()
    if "needs_layout_passes" in pltpu.CompilerParams.__dataclass_fields__:
        cp = dataclasses.replace(cp, needs_layout_passes=False)
    return pl.kernel(
        body, out_type=out_type, mesh=mesh, scratch_types=scratch,
        compiler_params=cp,
    )


def _make_proj(NP, D, BN):
    def body(a_ref, d_ref, w_ref, b_ref, o_ref):
        a = a_ref[0] + a_ref[1]
        deg = jnp.sum(d_ref[...], axis=0)[:, None]
        inv = 1.0 / jnp.maximum(deg, 1.0)
        h = jnp.dot(a * inv, w_ref[...], preferred_element_type=jnp.float32)
        o_ref[...] = jnp.maximum(h + b_ref[...], 0.0)

    return pl.pallas_call(
        body,
        grid=(NP // BN,),
        in_specs=[
            pl.BlockSpec((NC, BN, D), lambda i: (0, i, 0)),
            pl.BlockSpec((NW, BN), lambda i: (0, i)),
            pl.BlockSpec((D, D), lambda i: (0, 0)),
            pl.BlockSpec((1, D), lambda i: (0, 0)),
        ],
        out_specs=pl.BlockSpec((BN, D), lambda i: (i, 0)),
        out_shape=jax.ShapeDtypeStruct((NP, D), jnp.float32),
    )


@jax.jit
def kernel(x, edge_index, W1, b1, W2, b2):
    N, D = x.shape
    E = edge_index.shape[1]
    rows_pt = ((N + NS - 1) // NS + K - 1) // K * K
    NP = rows_pt * NS
    n_chunks = (E + NW * K - 1) // (NW * K)
    E_pad = NW * K * n_chunks

    src = edge_index[0].astype(jnp.int32)
    dst = edge_index[1].astype(jnp.int32)
    pad = E_pad - E
    if pad:
        src = jnp.concatenate([src, jnp.zeros((pad,), jnp.int32)])
        dst = jnp.concatenate([dst, jnp.full((pad,), N, jnp.int32)])

    xp = jnp.zeros((NP, D), jnp.float32).at[:N].set(x)

    agg_deg = _make_agg(NP, D, n_chunks, True)
    agg_only = _make_agg(NP, D, n_chunks, False)
    proj1 = _make_proj(NP, W1.shape[1], 1024)
    proj2 = _make_proj(NP, W2.shape[1], 1024)

    agg1, deg_flat = agg_deg(xp, src, dst)
    deg = deg_flat.reshape(NW, NP)
    h1 = proj1(agg1, deg, W1, b1.reshape(1, -1))
    (agg2,) = agg_only(h1, src, dst)
    h2 = proj2(agg2, deg, W2, b2.reshape(1, -1))
    return h2[:N]


# speedup vs baseline: 4.9911x; 1.1277x over previous
import dataclasses

import jax
import jax.numpy as jnp
from jax import lax
from jax.experimental import pallas as pl
from jax.experimental.pallas import tpu as pltpu
from jax.experimental.pallas import tpu_sc as plsc

NC = 2
NS = 16
NW = NC * NS
K = 128


def _make_agg(NP, D, n_chunks, compute_deg):
    rows_pt = NP // NS

    mesh = plsc.VectorSubcoreMesh(
        core_axis_name="c", subcore_axis_name="s", num_cores=NC, num_subcores=NS
    )

    BF = 8

    out_type = [jax.ShapeDtypeStruct((NC, NP, D), jnp.float32)]
    scratch = [
        pltpu.VMEM((K, D), jnp.float32),
        pltpu.VMEM((BF * K,), jnp.int32),
        pltpu.VMEM((K,), jnp.int32),
        pltpu.VMEM((K,), jnp.int32),
        pltpu.VMEM_SHARED((NP, D), jnp.float32),
    ]
    if compute_deg:
        out_type.append(jax.ShapeDtypeStruct((NW * NP,), jnp.float32))
        scratch.append(pltpu.VMEM((NP,), jnp.float32))

    def body(feat_hbm, pk_hbm, out_hbm, *rest):
        if compute_deg:
            deg_hbm = rest[0]
            rest = rest[1:]
        rows_v, pk_v, src_v, dst_v, agg_sh = rest[:5]
        if compute_deg:
            deg_v = rest[5]
        c = lax.axis_index("c")
        s = lax.axis_index("s")
        wid = c * NS + s
        base = s * rows_pt
        ebase = wid * (n_chunks * K)
        ones16 = jnp.ones((16,), jnp.float32)

        @pl.loop(0, K)
        def _(r):
            @pl.loop(0, D, step=16)
            def _(cc):
                rows_v[r, pl.ds(cc, 16)] = jnp.zeros((16,), jnp.float32)

        @pl.loop(0, rows_pt, step=K)
        def _(r0):
            pltpu.sync_copy(rows_v, agg_sh.at[pl.ds(base + r0, K)])

        if compute_deg:
            @pl.loop(0, NP, step=16)
            def _(r0):
                deg_v[pl.ds(r0, 16)] = jnp.zeros((16,), jnp.float32)

        plsc.subcore_barrier()

        @pl.loop(0, n_chunks)
        def _(j):
            poff = lax.bitwise_and(j, jnp.int32(BF - 1)) * K

            @pl.when(poff == 0)
            def _():
                pltpu.sync_copy(pk_hbm.at[pl.ds(ebase + j * K, BF * K)], pk_v)

            @pl.loop(0, K, step=16)
            def _(t):
                p = pk_v[pl.ds(poff + t, 16)]
                sv = lax.bitwise_and(p, jnp.int32(0xFFFF))
                dv = lax.shift_right_logical(p, jnp.int32(16))
                src_v[pl.ds(t, 16)] = sv
                dst_v[pl.ds(t, 16)] = dv
                if compute_deg:
                    plsc.addupdate_scatter(deg_v, [dv], ones16)

            pltpu.sync_copy(feat_hbm.at[src_v], rows_v)
            pltpu.sync_copy(rows_v, agg_sh.at[dst_v], add=True)

        plsc.subcore_barrier()
        pltpu.sync_copy(
            agg_sh.at[pl.ds(base, rows_pt)], out_hbm.at[c, pl.ds(base, rows_pt)]
        )
        if compute_deg:
            pltpu.sync_copy(deg_v, deg_hbm.at[pl.ds(wid * NP, NP)])

    cp = pltpu.CompilerParams()
    if "needs_layout_passes" in pltpu.CompilerParams.__dataclass_fields__:
        cp = dataclasses.replace(cp, needs_layout_passes=False)
    return pl.kernel(
        body, out_type=out_type, mesh=mesh, scratch_types=scratch,
        compiler_params=cp,
    )


def _make_proj(NP, D, BN):
    def body(a_ref, d_ref, w_ref, b_ref, o_ref):
        a = a_ref[0] + a_ref[1]
        deg = jnp.sum(d_ref[...], axis=0)[:, None]
        inv = 1.0 / jnp.maximum(deg, 1.0)
        h = jnp.dot(a * inv, w_ref[...], preferred_element_type=jnp.float32)
        o_ref[...] = jnp.maximum(h + b_ref[...], 0.0)

    return pl.pallas_call(
        body,
        grid=(NP // BN,),
        in_specs=[
            pl.BlockSpec((NC, BN, D), lambda i: (0, i, 0)),
            pl.BlockSpec((NW, BN), lambda i: (0, i)),
            pl.BlockSpec((D, D), lambda i: (0, 0)),
            pl.BlockSpec((1, D), lambda i: (0, 0)),
        ],
        out_specs=pl.BlockSpec((BN, D), lambda i: (i, 0)),
        out_shape=jax.ShapeDtypeStruct((NP, D), jnp.float32),
    )


@jax.jit
def kernel(x, edge_index, W1, b1, W2, b2):
    N, D = x.shape
    E = edge_index.shape[1]
    rows_pt = ((N + NS - 1) // NS + K - 1) // K * K
    NP = rows_pt * NS
    n_chunks = (E + NW * K - 1) // (NW * K)
    E_pad = NW * K * n_chunks

    src = edge_index[0].astype(jnp.int32)
    dst = edge_index[1].astype(jnp.int32)
    pad = E_pad - E
    if pad:
        src = jnp.concatenate([src, jnp.zeros((pad,), jnp.int32)])
        dst = jnp.concatenate([dst, jnp.full((pad,), N, jnp.int32)])
    packed = jnp.bitwise_or(jnp.left_shift(dst, 16), src)

    xp = jnp.zeros((NP, D), jnp.float32).at[:N].set(x)

    agg_deg = _make_agg(NP, D, n_chunks, True)
    agg_only = _make_agg(NP, D, n_chunks, False)
    proj1 = _make_proj(NP, W1.shape[1], 1024)
    proj2 = _make_proj(NP, W2.shape[1], 1024)

    agg1, deg_flat = agg_deg(xp, packed)
    deg = deg_flat.reshape(NW, NP)
    h1 = proj1(agg1, deg, W1, b1.reshape(1, -1))
    (agg2,) = agg_only(h1, packed)
    h2 = proj2(agg2, deg, W2, b2.reshape(1, -1))
    return h2[:N]


# speedup vs baseline: 5.0211x; 1.0060x over previous
import dataclasses

import jax
import jax.numpy as jnp
from jax import lax
from jax.experimental import pallas as pl
from jax.experimental.pallas import tpu as pltpu
from jax.experimental.pallas import tpu_sc as plsc

NC = 2
NS = 16
NW = NC * NS
K = 128


def _make_agg(NP, D, n_chunks, compute_deg):
    rows_pt = NP // NS

    mesh = plsc.VectorSubcoreMesh(
        core_axis_name="c", subcore_axis_name="s", num_cores=NC, num_subcores=NS
    )

    BF = 16

    out_type = [jax.ShapeDtypeStruct((NC, NP, D), jnp.float32)]
    scratch = [
        pltpu.VMEM((K, D), jnp.float32),
        pltpu.VMEM((BF * K,), jnp.int32),
        pltpu.VMEM((K,), jnp.int32),
        pltpu.VMEM((K,), jnp.int32),
        pltpu.VMEM_SHARED((NP, D), jnp.float32),
    ]
    if compute_deg:
        out_type.append(jax.ShapeDtypeStruct((NW * NP,), jnp.float32))
        scratch.append(pltpu.VMEM((NP,), jnp.float32))

    def body(feat_hbm, pk_hbm, out_hbm, *rest):
        if compute_deg:
            deg_hbm = rest[0]
            rest = rest[1:]
        rows_v, pk_v, src_v, dst_v, agg_sh = rest[:5]
        if compute_deg:
            deg_v = rest[5]
        c = lax.axis_index("c")
        s = lax.axis_index("s")
        wid = c * NS + s
        base = s * rows_pt
        ebase = wid * (n_chunks * K)
        ones16 = jnp.ones((16,), jnp.float32)

        @pl.loop(0, K)
        def _(r):
            @pl.loop(0, D, step=16)
            def _(cc):
                rows_v[r, pl.ds(cc, 16)] = jnp.zeros((16,), jnp.float32)

        @pl.loop(0, rows_pt, step=K)
        def _(r0):
            pltpu.sync_copy(rows_v, agg_sh.at[pl.ds(base + r0, K)])

        if compute_deg:
            @pl.loop(0, NP, step=16)
            def _(r0):
                deg_v[pl.ds(r0, 16)] = jnp.zeros((16,), jnp.float32)

        plsc.subcore_barrier()

        @pl.loop(0, n_chunks)
        def _(j):
            poff = lax.bitwise_and(j, jnp.int32(BF - 1)) * K

            @pl.when(poff == 0)
            def _():
                pltpu.sync_copy(pk_hbm.at[pl.ds(ebase + j * K, BF * K)], pk_v)

            @pl.loop(0, K, step=16)
            def _(t):
                p = pk_v[pl.ds(poff + t, 16)]
                sv = lax.bitwise_and(p, jnp.int32(0xFFFF))
                dv = lax.shift_right_logical(p, jnp.int32(16))
                src_v[pl.ds(t, 16)] = sv
                dst_v[pl.ds(t, 16)] = dv
                if compute_deg:
                    plsc.addupdate_scatter(deg_v, [dv], ones16)

            pltpu.sync_copy(feat_hbm.at[src_v], rows_v)
            pltpu.sync_copy(rows_v, agg_sh.at[dst_v], add=True)

        plsc.subcore_barrier()
        pltpu.sync_copy(
            agg_sh.at[pl.ds(base, rows_pt)], out_hbm.at[c, pl.ds(base, rows_pt)]
        )
        if compute_deg:
            pltpu.sync_copy(deg_v, deg_hbm.at[pl.ds(wid * NP, NP)])

    cp = pltpu.CompilerParams()
    if "needs_layout_passes" in pltpu.CompilerParams.__dataclass_fields__:
        cp = dataclasses.replace(cp, needs_layout_passes=False)
    return pl.kernel(
        body, out_type=out_type, mesh=mesh, scratch_types=scratch,
        compiler_params=cp,
    )


def _make_proj(NP, D, BN):
    def body(a_ref, d_ref, w_ref, b_ref, o_ref):
        a = a_ref[0] + a_ref[1]
        deg = jnp.sum(d_ref[...], axis=0)[:, None]
        inv = 1.0 / jnp.maximum(deg, 1.0)
        h = jnp.dot(a * inv, w_ref[...], preferred_element_type=jnp.float32)
        o_ref[...] = jnp.maximum(h + b_ref[...], 0.0)

    return pl.pallas_call(
        body,
        grid=(NP // BN,),
        in_specs=[
            pl.BlockSpec((NC, BN, D), lambda i: (0, i, 0)),
            pl.BlockSpec((NW, BN), lambda i: (0, i)),
            pl.BlockSpec((D, D), lambda i: (0, 0)),
            pl.BlockSpec((1, D), lambda i: (0, 0)),
        ],
        out_specs=pl.BlockSpec((BN, D), lambda i: (i, 0)),
        out_shape=jax.ShapeDtypeStruct((NP, D), jnp.float32),
    )


@jax.jit
def kernel(x, edge_index, W1, b1, W2, b2):
    N, D = x.shape
    E = edge_index.shape[1]
    rows_pt = ((N + NS - 1) // NS + K - 1) // K * K
    NP = rows_pt * NS
    n_chunks = (E + NW * K - 1) // (NW * K)
    E_pad = NW * K * n_chunks

    src = edge_index[0].astype(jnp.int32)
    dst = edge_index[1].astype(jnp.int32)
    pad = E_pad - E
    if pad:
        src = jnp.concatenate([src, jnp.zeros((pad,), jnp.int32)])
        dst = jnp.concatenate([dst, jnp.full((pad,), N, jnp.int32)])
    packed = jnp.bitwise_or(jnp.left_shift(dst, 16), src)

    xp = jnp.zeros((NP, D), jnp.float32).at[:N].set(x)

    agg_deg = _make_agg(NP, D, n_chunks, True)
    agg_only = _make_agg(NP, D, n_chunks, False)
    proj1 = _make_proj(NP, W1.shape[1], 1024)
    proj2 = _make_proj(NP, W2.shape[1], 1024)

    agg1, deg_flat = agg_deg(xp, packed)
    deg = deg_flat.reshape(NW, NP)
    h1 = proj1(agg1, deg, W1, b1.reshape(1, -1))
    (agg2,) = agg_only(h1, packed)
    h2 = proj2(agg2, deg, W2, b2.reshape(1, -1))
    return h2[:N]
